# baseline (speedup 1.0000x reference)
import functools

import jax
import jax.numpy as jnp
from jax import lax
from jax.experimental import pallas as pl
from jax.experimental.pallas import tpu as pltpu
from jax.experimental.pallas import tpu_sc as plsc

N_NODES = 10000
D_FEAT = 128
HIDDEN = 16
N_CLASSES = 16

NC, NS, L = 2, 16, 16
NW = NC * NS
NP = 10240
RPS = NP // NS
TRASH = N_NODES

E = 320000
EPW = E // NW
GLEN = 512
NGRP = 20
EPWP = NGRP * GLEN
PADW = EPWP - EPW
NBUF = 8

BLK = 1024
GRID = 10

_mesh = plsc.VectorSubcoreMesh(
    core_axis_name="c", subcore_axis_name="s", num_cores=NC, num_subcores=NS
)
_sc_params = pltpu.CompilerParams(use_tc_tiling_on_sc=False)


@functools.partial(
    pl.kernel,
    out_type=[
        jax.ShapeDtypeStruct((NP,), jnp.float32),
        jax.ShapeDtypeStruct((NP,), jnp.float32),
    ],
    mesh=_mesh,
    scratch_types=[
        pltpu.VMEM((EPWP,), jnp.int32),
        pltpu.VMEM((GLEN,), jnp.float32),
        pltpu.VMEM_SHARED((NP,), jnp.float32),
        pltpu.SemaphoreType.DMA,
    ],
    compiler_params=_sc_params,
)
def _deg_kernel(dst_hbm, zeros_hbm, out0_hbm, out1_hbm, dst_v, ones_v,
                deg_sp, sem):
    c = lax.axis_index("c")
    s = lax.axis_index("s")
    wid = c * NS + s
    sl = pl.ds(s * RPS, RPS)
    for i in range(GLEN // L):
        ones_v[pl.ds(i * L, L)] = jnp.full((L,), 1.0, jnp.float32)
    pltpu.sync_copy(zeros_hbm.at[sl], deg_sp.at[sl])
    pltpu.sync_copy(dst_hbm.at[pl.ds(wid * EPWP, EPWP)], dst_v)
    plsc.subcore_barrier()

    def body(j, carry):
        pltpu.async_copy(ones_v, deg_sp.at[dst_v.at[pl.ds(j * GLEN, GLEN)]],
                         sem, add=True)
        return carry

    lax.fori_loop(0, NGRP, body, 0)

    def drain(j, carry):
        pltpu.make_async_copy(ones_v, deg_sp.at[dst_v.at[pl.ds(0, GLEN)]],
                              sem).wait()
        return carry

    lax.fori_loop(0, NGRP, drain, 0)
    plsc.subcore_barrier()

    @pl.when(c == 0)
    def _():
        pltpu.sync_copy(deg_sp.at[sl], out0_hbm.at[sl])

    @pl.when(c == 1)
    def _():
        pltpu.sync_copy(deg_sp.at[sl], out1_hbm.at[sl])


@functools.partial(
    pl.kernel,
    out_type=jax.ShapeDtypeStruct((NC, NP, HIDDEN), jnp.float32),
    mesh=_mesh,
    scratch_types=[
        pltpu.VMEM((EPWP,), jnp.int32),
        pltpu.VMEM((EPWP,), jnp.int32),
        pltpu.VMEM((NBUF, GLEN, HIDDEN), jnp.float32),
        pltpu.VMEM_SHARED((NP, HIDDEN), jnp.float32),
        pltpu.VMEM_SHARED((N_NODES, HIDDEN), jnp.float32),
        pltpu.SemaphoreType.DMA,
    ],
    compiler_params=_sc_params,
)
def _agg_kernel(src_hbm, dst_hbm, table_hbm, zeros_hbm, out_hbm,
                src_v, dst_v, rows_v, agg_sp, table_sp, gsem):
    c = lax.axis_index("c")
    s = lax.axis_index("s")
    wid = c * NS + s
    sl = pl.ds(s * RPS, RPS)
    pltpu.sync_copy(zeros_hbm.at[sl], agg_sp.at[sl])
    tsl = pl.ds(s * (N_NODES // NS), N_NODES // NS)
    pltpu.sync_copy(table_hbm.at[tsl], table_sp.at[tsl])
    pltpu.sync_copy(src_hbm.at[pl.ds(wid * EPWP, EPWP)], src_v)
    pltpu.sync_copy(dst_hbm.at[pl.ds(wid * EPWP, EPWP)], dst_v)
    plsc.subcore_barrier()

    for b in range(NBUF):
        pltpu.async_copy(table_sp.at[src_v.at[pl.ds(b * GLEN, GLEN)]],
                         rows_v.at[b], gsem)

    def body(t, carry):
        b = lax.rem(t, NBUF)
        pltpu.make_async_copy(table_sp.at[src_v.at[pl.ds(0, GLEN)]],
                              rows_v.at[0], gsem).wait()
        pltpu.sync_copy(rows_v.at[b],
                        agg_sp.at[dst_v.at[pl.ds(t * GLEN, GLEN)]], add=True)
        nxt = t + NBUF

        @pl.when(nxt < NGRP)
        def _():
            pltpu.async_copy(table_sp.at[src_v.at[pl.ds(nxt * GLEN, GLEN)]],
                             rows_v.at[b], gsem)

        return carry

    lax.fori_loop(0, NGRP, body, 0)
    plsc.subcore_barrier()
    pltpu.sync_copy(agg_sp.at[sl], out_hbm.at[c, sl])


def _dis_blk(d0_ref, d1_ref, i):
    deg = d0_ref[pl.ds(i * BLK, BLK)] + d1_ref[pl.ds(i * BLK, BLK)] + 1.0
    return lax.rsqrt(deg)[:, None]


def _l1_body(x_ref, w1_ref, d0_ref, d1_ref, ht_ref):
    i = pl.program_id(0)
    h = jnp.dot(x_ref[...], w1_ref[...], preferred_element_type=jnp.float32)
    ht_ref[...] = h * _dis_blk(d0_ref, d1_ref, i)


def _lin1(x, W1, deg0, deg1):
    return pl.pallas_call(
        _l1_body,
        grid=(GRID,),
        in_specs=[
            pl.BlockSpec((BLK, D_FEAT), lambda i: (i, 0)),
            pl.BlockSpec((D_FEAT, HIDDEN), lambda i: (0, 0)),
            pl.BlockSpec((NP,), lambda i: (0,)),
            pl.BlockSpec((NP,), lambda i: (0,)),
        ],
        out_specs=pl.BlockSpec((BLK, HIDDEN), lambda i: (i, 0)),
        out_shape=jax.ShapeDtypeStruct((N_NODES, HIDDEN), jnp.float32),
    )(x, W1, deg0, deg1)


def _l2_body(aggp_ref, ht_ref, d0_ref, d1_ref, w2_ref, b1_ref, ht2_ref,
             agg_v, sem):
    i = pl.program_id(0)
    pltpu.make_async_copy(
        aggp_ref.at[:, pl.ds(i * BLK, BLK)], agg_v, sem
    ).start()
    dis = _dis_blk(d0_ref, d1_ref, i)
    pltpu.make_async_copy(
        aggp_ref.at[:, pl.ds(i * BLK, BLK)], agg_v, sem
    ).wait()
    agg = agg_v[0, :, :] + agg_v[1, :, :] + ht_ref[...]
    z = jnp.maximum(agg * dis + b1_ref[...], 0.0)
    h2 = jnp.dot(z, w2_ref[...], preferred_element_type=jnp.float32)
    ht2_ref[...] = h2 * dis


def _lin2(aggp, ht1, deg0, deg1, W2, b1):
    return pl.pallas_call(
        _l2_body,
        grid=(GRID,),
        in_specs=[
            pl.BlockSpec(memory_space=pltpu.MemorySpace.HBM),
            pl.BlockSpec((BLK, HIDDEN), lambda i: (i, 0)),
            pl.BlockSpec((NP,), lambda i: (0,)),
            pl.BlockSpec((NP,), lambda i: (0,)),
            pl.BlockSpec((HIDDEN, N_CLASSES), lambda i: (0, 0)),
            pl.BlockSpec((1, N_CLASSES), lambda i: (0, 0)),
        ],
        out_specs=pl.BlockSpec((BLK, N_CLASSES), lambda i: (i, 0)),
        out_shape=jax.ShapeDtypeStruct((N_NODES, N_CLASSES), jnp.float32),
        scratch_shapes=[
            pltpu.VMEM((NC, BLK, HIDDEN), jnp.float32),
            pltpu.SemaphoreType.DMA,
        ],
    )(aggp, ht1, deg0, deg1, W2, b1)


def _fin_body(aggp_ref, ht2_ref, d0_ref, d1_ref, b2_ref, out_ref):
    i = pl.program_id(0)
    dis = _dis_blk(d0_ref, d1_ref, i)
    agg = aggp_ref[0, :, :] + aggp_ref[1, :, :] + ht2_ref[...]
    o = agg * dis + b2_ref[...]
    m = jnp.max(o, axis=1, keepdims=True)
    lse = jnp.log(jnp.sum(jnp.exp(o - m), axis=1, keepdims=True)) + m
    out_ref[...] = o - lse


def _final(aggp, ht2, deg0, deg1, b2):
    return pl.pallas_call(
        _fin_body,
        grid=(GRID,),
        in_specs=[
            pl.BlockSpec((NC, BLK, N_CLASSES), lambda i: (0, i, 0)),
            pl.BlockSpec((BLK, N_CLASSES), lambda i: (i, 0)),
            pl.BlockSpec((NP,), lambda i: (0,)),
            pl.BlockSpec((NP,), lambda i: (0,)),
            pl.BlockSpec((1, N_CLASSES), lambda i: (0, 0)),
        ],
        out_specs=pl.BlockSpec((BLK, N_CLASSES), lambda i: (i, 0)),
        out_shape=jax.ShapeDtypeStruct((N_NODES, N_CLASSES), jnp.float32),
    )(aggp, ht2, deg0, deg1, b2)


def kernel(x, edge_index, W1, b1, W2, b2):
    src = edge_index[0].reshape(NW, EPW)
    dst = edge_index[1].reshape(NW, EPW)
    src1 = jnp.concatenate(
        [src, jnp.zeros((NW, PADW), jnp.int32)], axis=1
    ).reshape(NW * EPWP)
    dst1 = jnp.concatenate(
        [dst, jnp.full((NW, PADW), TRASH, jnp.int32)], axis=1
    ).reshape(NW * EPWP)
    zeros_vec = jnp.zeros((NP,), jnp.float32)
    zeros_tab = jnp.zeros((NP, HIDDEN), jnp.float32)

    deg0, deg1 = _deg_kernel(dst1, zeros_vec)
    ht1 = _lin1(x, W1, deg0, deg1)
    agg1 = _agg_kernel(src1, dst1, ht1, zeros_tab)
    ht2 = _lin2(agg1, ht1, deg0, deg1, W2, b1.reshape(1, HIDDEN))
    agg2 = _agg_kernel(src1, dst1, ht2, zeros_tab)
    return _final(agg2, ht2, deg0, deg1, b2.reshape(1, N_CLASSES))


# speedup vs baseline: 1.0856x; 1.0856x over previous
import functools

import jax
import jax.numpy as jnp
from jax import lax
from jax.experimental import pallas as pl
from jax.experimental.pallas import tpu as pltpu
from jax.experimental.pallas import tpu_sc as plsc

N_NODES = 10000
D_FEAT = 128
HIDDEN = 16
N_CLASSES = 16

NC, NS, L = 2, 16, 16
NW = NC * NS
NP = 10240
RPS = NP // NS
TRASH = N_NODES

E = 320000
EPW = E // NW
GLEN = 1024
NGRP = 10
EPWP = NGRP * GLEN
PADW = EPWP - EPW
NBUF = 4

BLK = 1024
GRID = 10

_mesh = plsc.VectorSubcoreMesh(
    core_axis_name="c", subcore_axis_name="s", num_cores=NC, num_subcores=NS
)
_sc_params = pltpu.CompilerParams(use_tc_tiling_on_sc=False)


@functools.partial(
    pl.kernel,
    out_type=[
        jax.ShapeDtypeStruct((NP,), jnp.float32),
        jax.ShapeDtypeStruct((NP,), jnp.float32),
    ],
    mesh=_mesh,
    scratch_types=[
        pltpu.VMEM((EPWP,), jnp.int32),
        pltpu.VMEM((GLEN,), jnp.float32),
        pltpu.VMEM_SHARED((NP,), jnp.float32),
        pltpu.SemaphoreType.DMA,
    ],
    compiler_params=_sc_params,
)
def _deg_kernel(dst_hbm, zeros_hbm, out0_hbm, out1_hbm, dst_v, ones_v,
                deg_sp, sem):
    c = lax.axis_index("c")
    s = lax.axis_index("s")
    wid = c * NS + s
    sl = pl.ds(s * RPS, RPS)
    pltpu.async_copy(zeros_hbm.at[sl], deg_sp.at[sl], sem)
    pltpu.async_copy(dst_hbm.at[pl.ds(wid * EPWP, EPWP)], dst_v, sem)
    for i in range(GLEN // L):
        ones_v[pl.ds(i * L, L)] = jnp.full((L,), 1.0, jnp.float32)
    pltpu.make_async_copy(zeros_hbm.at[sl], deg_sp.at[sl], sem).wait()
    pltpu.make_async_copy(dst_hbm.at[pl.ds(wid * EPWP, EPWP)], dst_v,
                          sem).wait()
    plsc.subcore_barrier()

    def body(j, carry):
        pltpu.async_copy(ones_v, deg_sp.at[dst_v.at[pl.ds(j * GLEN, GLEN)]],
                         sem, add=True)
        return carry

    lax.fori_loop(0, NGRP, body, 0)

    def drain(j, carry):
        pltpu.make_async_copy(ones_v, deg_sp.at[dst_v.at[pl.ds(0, GLEN)]],
                              sem).wait()
        return carry

    lax.fori_loop(0, NGRP, drain, 0)
    plsc.subcore_barrier()

    @pl.when(c == 0)
    def _():
        pltpu.sync_copy(deg_sp.at[sl], out0_hbm.at[sl])

    @pl.when(c == 1)
    def _():
        pltpu.sync_copy(deg_sp.at[sl], out1_hbm.at[sl])


@functools.partial(
    pl.kernel,
    out_type=jax.ShapeDtypeStruct((NC, NP, HIDDEN), jnp.float32),
    mesh=_mesh,
    scratch_types=[
        pltpu.VMEM((EPWP,), jnp.int32),
        pltpu.VMEM((EPWP,), jnp.int32),
        pltpu.VMEM((NBUF, GLEN, HIDDEN), jnp.float32),
        pltpu.VMEM_SHARED((NP, HIDDEN), jnp.float32),
        pltpu.VMEM_SHARED((N_NODES, HIDDEN), jnp.float32),
        pltpu.SemaphoreType.DMA,
    ],
    compiler_params=_sc_params,
)
def _agg_kernel(src_hbm, dst_hbm, table_hbm, zeros_hbm, out_hbm,
                src_v, dst_v, rows_v, agg_sp, table_sp, gsem):
    c = lax.axis_index("c")
    s = lax.axis_index("s")
    wid = c * NS + s
    sl = pl.ds(s * RPS, RPS)
    tsl = pl.ds(s * (N_NODES // NS), N_NODES // NS)
    pltpu.async_copy(zeros_hbm.at[sl], agg_sp.at[sl], gsem)
    pltpu.async_copy(table_hbm.at[tsl], table_sp.at[tsl], gsem)
    pltpu.async_copy(src_hbm.at[pl.ds(wid * EPWP, EPWP)], src_v, gsem)
    pltpu.async_copy(dst_hbm.at[pl.ds(wid * EPWP, EPWP)], dst_v, gsem)
    pltpu.make_async_copy(zeros_hbm.at[sl], agg_sp.at[sl], gsem).wait()
    pltpu.make_async_copy(table_hbm.at[tsl], table_sp.at[tsl], gsem).wait()
    pltpu.make_async_copy(src_hbm.at[pl.ds(wid * EPWP, EPWP)], src_v,
                          gsem).wait()
    pltpu.make_async_copy(dst_hbm.at[pl.ds(wid * EPWP, EPWP)], dst_v,
                          gsem).wait()
    plsc.subcore_barrier()

    for b in range(NBUF):
        pltpu.async_copy(table_sp.at[src_v.at[pl.ds(b * GLEN, GLEN)]],
                         rows_v.at[b], gsem)

    def body(t, carry):
        b = lax.rem(t, NBUF)
        pltpu.make_async_copy(table_sp.at[src_v.at[pl.ds(0, GLEN)]],
                              rows_v.at[0], gsem).wait()
        pltpu.sync_copy(rows_v.at[b],
                        agg_sp.at[dst_v.at[pl.ds(t * GLEN, GLEN)]], add=True)
        nxt = t + NBUF

        @pl.when(nxt < NGRP)
        def _():
            pltpu.async_copy(table_sp.at[src_v.at[pl.ds(nxt * GLEN, GLEN)]],
                             rows_v.at[b], gsem)

        return carry

    lax.fori_loop(0, NGRP, body, 0)
    plsc.subcore_barrier()
    pltpu.sync_copy(agg_sp.at[sl], out_hbm.at[c, sl])


def _dis_blk(d0_ref, d1_ref, i):
    deg = d0_ref[pl.ds(i * BLK, BLK)] + d1_ref[pl.ds(i * BLK, BLK)] + 1.0
    return lax.rsqrt(deg)[:, None]


def _l1_body(x_ref, w1_ref, d0_ref, d1_ref, ht_ref):
    i = pl.program_id(0)
    h = jnp.dot(x_ref[...], w1_ref[...], preferred_element_type=jnp.float32)
    ht_ref[...] = h * _dis_blk(d0_ref, d1_ref, i)


def _lin1(x, W1, deg0, deg1):
    return pl.pallas_call(
        _l1_body,
        grid=(GRID,),
        in_specs=[
            pl.BlockSpec((BLK, D_FEAT), lambda i: (i, 0)),
            pl.BlockSpec((D_FEAT, HIDDEN), lambda i: (0, 0)),
            pl.BlockSpec((NP,), lambda i: (0,)),
            pl.BlockSpec((NP,), lambda i: (0,)),
        ],
        out_specs=pl.BlockSpec((BLK, HIDDEN), lambda i: (i, 0)),
        out_shape=jax.ShapeDtypeStruct((N_NODES, HIDDEN), jnp.float32),
    )(x, W1, deg0, deg1)


def _l2_body(aggp_ref, ht_ref, d0_ref, d1_ref, w2_ref, b1_ref, ht2_ref):
    i = pl.program_id(0)
    dis = _dis_blk(d0_ref, d1_ref, i)
    agg = aggp_ref[0, :, :] + aggp_ref[1, :, :] + ht_ref[...]
    z = jnp.maximum(agg * dis + b1_ref[...], 0.0)
    h2 = jnp.dot(z, w2_ref[...], preferred_element_type=jnp.float32)
    ht2_ref[...] = h2 * dis


def _lin2(aggp, ht1, deg0, deg1, W2, b1):
    return pl.pallas_call(
        _l2_body,
        grid=(GRID,),
        in_specs=[
            pl.BlockSpec((NC, BLK, HIDDEN), lambda i: (0, i, 0)),
            pl.BlockSpec((BLK, HIDDEN), lambda i: (i, 0)),
            pl.BlockSpec((NP,), lambda i: (0,)),
            pl.BlockSpec((NP,), lambda i: (0,)),
            pl.BlockSpec((HIDDEN, N_CLASSES), lambda i: (0, 0)),
            pl.BlockSpec((1, N_CLASSES), lambda i: (0, 0)),
        ],
        out_specs=pl.BlockSpec((BLK, N_CLASSES), lambda i: (i, 0)),
        out_shape=jax.ShapeDtypeStruct((N_NODES, N_CLASSES), jnp.float32),
    )(aggp, ht1, deg0, deg1, W2, b1)


def _fin_body(aggp_ref, ht2_ref, d0_ref, d1_ref, b2_ref, out_ref):
    i = pl.program_id(0)
    dis = _dis_blk(d0_ref, d1_ref, i)
    agg = aggp_ref[0, :, :] + aggp_ref[1, :, :] + ht2_ref[...]
    o = agg * dis + b2_ref[...]
    m = jnp.max(o, axis=1, keepdims=True)
    lse = jnp.log(jnp.sum(jnp.exp(o - m), axis=1, keepdims=True)) + m
    out_ref[...] = o - lse


def _final(aggp, ht2, deg0, deg1, b2):
    return pl.pallas_call(
        _fin_body,
        grid=(GRID,),
        in_specs=[
            pl.BlockSpec((NC, BLK, N_CLASSES), lambda i: (0, i, 0)),
            pl.BlockSpec((BLK, N_CLASSES), lambda i: (i, 0)),
            pl.BlockSpec((NP,), lambda i: (0,)),
            pl.BlockSpec((NP,), lambda i: (0,)),
            pl.BlockSpec((1, N_CLASSES), lambda i: (0, 0)),
        ],
        out_specs=pl.BlockSpec((BLK, N_CLASSES), lambda i: (i, 0)),
        out_shape=jax.ShapeDtypeStruct((N_NODES, N_CLASSES), jnp.float32),
    )(aggp, ht2, deg0, deg1, b2)


def kernel(x, edge_index, W1, b1, W2, b2):
    src = edge_index[0].reshape(NW, EPW)
    dst = edge_index[1].reshape(NW, EPW)
    src1 = jnp.concatenate(
        [src, jnp.zeros((NW, PADW), jnp.int32)], axis=1
    ).reshape(NW * EPWP)
    dst1 = jnp.concatenate(
        [dst, jnp.full((NW, PADW), TRASH, jnp.int32)], axis=1
    ).reshape(NW * EPWP)
    zeros_vec = jnp.zeros((NP,), jnp.float32)
    zeros_tab = jnp.zeros((NP, HIDDEN), jnp.float32)

    deg0, deg1 = _deg_kernel(dst1, zeros_vec)
    ht1 = _lin1(x, W1, deg0, deg1)
    agg1 = _agg_kernel(src1, dst1, ht1, zeros_tab)
    ht2 = _lin2(agg1, ht1, deg0, deg1, W2, b1.reshape(1, HIDDEN))
    agg2 = _agg_kernel(src1, dst1, ht2, zeros_tab)
    return _final(agg2, ht2, deg0, deg1, b2.reshape(1, N_CLASSES))
